# baseline (speedup 1.0000x reference)
import jax
import jax.numpy as jnp
from jax.experimental import pallas as pl
from jax.experimental.pallas import tpu as pltpu

_T = 8192
_D = 4096
_E = 64
_TOP_K = 8
_BT = 1024


def _router_body(x_ref, w_ref, wout_ref, iout_ref):
    x = x_ref[...]
    w = w_ref[...]
    logits = jnp.dot(x, w, preferred_element_type=jnp.float32)

    coli = jax.lax.broadcasted_iota(jnp.int32, (_BT, _E), 1)
    bits = jax.lax.bitcast_convert_type(logits, jnp.int32)
    key_bits = (bits & -64) | (63 - coli)
    key = jax.lax.bitcast_convert_type(key_bits, jnp.float32)

    picked = []
    for _ in range(_TOP_K):
        m = jnp.max(key, axis=1, keepdims=True)
        picked.append(m)
        key = jnp.where(key == m, -jnp.inf, key)

    kcat = jnp.concatenate(picked, axis=1)
    kcat_bits = jax.lax.bitcast_convert_type(kcat, jnp.int32)
    iout_ref[...] = 63 - (kcat_bits & 63)
    v = jax.lax.bitcast_convert_type(kcat_bits & -64, jnp.float32)
    e = jnp.exp(v - v[:, 0:1])
    wout_ref[...] = e / jnp.sum(e, axis=1, keepdims=True)


@jax.jit
def kernel(x_TD, kernel_DE):
    x_TD = jnp.asarray(x_TD, jnp.float32)
    grid = (_T // _BT,)
    wout, iout = pl.pallas_call(
        _router_body,
        grid=grid,
        in_specs=[
            pl.BlockSpec((_BT, _D), lambda i: (i, 0)),
            pl.BlockSpec((_D, _E), lambda i: (0, 0)),
        ],
        out_specs=[
            pl.BlockSpec((_BT, _TOP_K), lambda i: (i, 0)),
            pl.BlockSpec((_BT, _TOP_K), lambda i: (i, 0)),
        ],
        out_shape=[
            jax.ShapeDtypeStruct((_T, _TOP_K), jnp.float32),
            jax.ShapeDtypeStruct((_T, _TOP_K), jnp.int32),
        ],
        compiler_params=pltpu.CompilerParams(
            dimension_semantics=("parallel",),
        ),
    )(x_TD, kernel_DE)
    return wout, iout
